# Initial kernel scaffold; baseline (speedup 1.0000x reference)
import jax
import jax.numpy as jnp
from jax.experimental import pallas as pl


def kernel(logits, labels):
    raise NotImplementedError("write your pallas kernel here")


# speedup vs baseline: 2.5997x; 2.5997x over previous
import functools

import jax
import jax.numpy as jnp
from jax.experimental import pallas as pl
from jax.experimental.pallas import tpu as pltpu

R = 2048
C = 128
N = R * C
LOGN = 18
B = 8


def _loss_kernel(logits_ref, labels_ref, out_ref):
    lg = logits_ref[...].reshape(R, C)
    lb = labels_ref[...].reshape(R, C)
    lbf = lb.astype(jnp.float32)
    signs = 2.0 * lbf - 1.0
    e = 1.0 - lg * signs

    maxe = jnp.max(e)

    bits = jax.lax.bitcast_convert_type(e, jnp.int32)
    ikey = bits ^ ((bits >> 31) & jnp.int32(0x7FFFFFFF))
    x = (ikey & jnp.int32(~1)) | lb

    idx = (jax.lax.broadcasted_iota(jnp.int32, (R, C), 1) * R
           + jax.lax.broadcasted_iota(jnp.int32, (R, C), 0))

    for kk in range(1, LOGN + 1):
        desc = ((idx >> kk) & 1) == 0
        for j in range(kk - 1, -1, -1):
            s = 1 << j
            is_lower = (idx & s) == 0
            if s < R:
                down = pltpu.roll(x, R - s, axis=0)
                up = pltpu.roll(x, s, axis=0)
            else:
                t = s // R
                down = pltpu.roll(x, C - t, axis=1)
                up = pltpu.roll(x, t, axis=1)
            partner = jnp.where(is_lower, down, up)
            hi = jnp.maximum(x, partner)
            lo = jnp.minimum(x, partner)
            x = jnp.where(is_lower == desc, hi, lo)

    gt = (x & 1).astype(jnp.float32)
    bdec = jnp.where(x >= 0, x, x ^ jnp.int32(0x7FFFFFFF))
    e_s = jax.lax.bitcast_convert_type(bdec, jnp.float32)
    relu_e = jnp.maximum(e_s, 0.0)

    col = gt
    riota = jax.lax.broadcasted_iota(jnp.int32, (R, C), 0)
    d = 1
    while d < R:
        shifted = pltpu.roll(col, d, axis=0)
        col = col + jnp.where(riota >= d, shifted, 0.0)
        d *= 2
    colsum = jax.lax.slice(col, (R - 1, 0), (R, C))
    liota = jax.lax.broadcasted_iota(jnp.int32, (1, C), 1)
    lane = colsum
    d = 1
    while d < C:
        shifted = pltpu.roll(lane, d, axis=1)
        lane = lane + jnp.where(liota >= d, shifted, 0.0)
        d *= 2
    G = jax.lax.slice(lane, (0, C - 1), (1, C))[0, 0]
    lane_ex = lane - colsum
    c = col + lane_ex

    kpos = idx.astype(jnp.float32)
    U = G + (kpos + 1.0) - c
    Um1 = jnp.maximum(U - 1.0 + gt, 1.0)
    grad = gt / U + (1.0 - gt) * (G - c) / (U * Um1)
    loss = jnp.sum(relu_e * grad)
    loss = jnp.where(G > 0.0, loss, jnp.maximum(maxe, 0.0))
    out_ref[...] = jnp.full((1, 1, C), loss, dtype=jnp.float32)


@jax.jit
def kernel(logits, labels):
    lg = logits.reshape(B, R, C)
    lb = labels.reshape(B, R, C)
    out = pl.pallas_call(
        _loss_kernel,
        grid=(B,),
        in_specs=[
            pl.BlockSpec((1, R, C), lambda i: (i, 0, 0)),
            pl.BlockSpec((1, R, C), lambda i: (i, 0, 0)),
        ],
        out_specs=pl.BlockSpec((1, 1, C), lambda i: (i, 0, 0)),
        out_shape=jax.ShapeDtypeStruct((B, 1, C), jnp.float32),
    )(lg, lb)
    return jnp.mean(out[:, 0, 0])


# speedup vs baseline: 3.5333x; 1.3591x over previous
import functools

import jax
import jax.numpy as jnp
from jax.experimental import pallas as pl
from jax.experimental.pallas import tpu as pltpu

R = 2048
C = 128
N = R * C
LOGN = 18
B = 8


def _loss_kernel(logits_ref, labels_ref, out_ref):
    lg = logits_ref[...].reshape(R, C)
    lb = labels_ref[...].reshape(R, C)
    lbf = lb.astype(jnp.float32)
    signs = 2.0 * lbf - 1.0
    e = 1.0 - lg * signs

    maxe = jnp.max(e)

    bits = jax.lax.bitcast_convert_type(e, jnp.int32)
    ikey = bits ^ ((bits >> 31) & jnp.int32(0x7FFFFFFF))
    x = (ikey & jnp.int32(~1)) | lb

    idx = (jax.lax.broadcasted_iota(jnp.int32, (R, C), 1) * R
           + jax.lax.broadcasted_iota(jnp.int32, (R, C), 0))

    for kk in range(1, LOGN + 1):
        desc = ((idx >> kk) & 1) == 0
        for j in range(kk - 1, -1, -1):
            s = 1 << j
            if 8 <= s < R:
                O = R // (2 * s)
                y = x.reshape(O, 2, s, C)
                a = y[:, 0, :, :]
                b = y[:, 1, :, :]
                hi = jnp.maximum(a, b)
                lo = jnp.minimum(a, b)
                if kk <= 10:
                    pos = jax.lax.broadcasted_iota(jnp.int32, (O, 1, C), 0) * (2 * s)
                else:
                    pos = jax.lax.broadcasted_iota(jnp.int32, (O, 1, C), 2) * R
                dsc = ((pos >> kk) & 1) == 0
                na = jnp.where(dsc, hi, lo)
                nb = jnp.where(dsc, lo, hi)
                x = jnp.concatenate([na[:, None], nb[:, None]], axis=1).reshape(R, C)
            else:
                is_lower = (idx & s) == 0
                if s < R:
                    down = pltpu.roll(x, R - s, axis=0)
                    up = pltpu.roll(x, s, axis=0)
                else:
                    t = s // R
                    down = pltpu.roll(x, C - t, axis=1)
                    up = pltpu.roll(x, t, axis=1)
                partner = jnp.where(is_lower, down, up)
                hi = jnp.maximum(x, partner)
                lo = jnp.minimum(x, partner)
                x = jnp.where(is_lower == desc, hi, lo)

    gt = (x & 1).astype(jnp.float32)
    bdec = jnp.where(x >= 0, x, x ^ jnp.int32(0x7FFFFFFF))
    e_s = jax.lax.bitcast_convert_type(bdec, jnp.float32)
    relu_e = jnp.maximum(e_s, 0.0)

    col = gt
    riota = jax.lax.broadcasted_iota(jnp.int32, (R, C), 0)
    d = 1
    while d < R:
        shifted = pltpu.roll(col, d, axis=0)
        col = col + jnp.where(riota >= d, shifted, 0.0)
        d *= 2
    colsum = jax.lax.slice(col, (R - 1, 0), (R, C))
    liota = jax.lax.broadcasted_iota(jnp.int32, (1, C), 1)
    lane = colsum
    d = 1
    while d < C:
        shifted = pltpu.roll(lane, d, axis=1)
        lane = lane + jnp.where(liota >= d, shifted, 0.0)
        d *= 2
    G = jax.lax.slice(lane, (0, C - 1), (1, C))[0, 0]
    lane_ex = lane - colsum
    c = col + lane_ex

    kpos = idx.astype(jnp.float32)
    U = G + (kpos + 1.0) - c
    Um1 = jnp.maximum(U - 1.0 + gt, 1.0)
    grad = gt / U + (1.0 - gt) * (G - c) / (U * Um1)
    loss = jnp.sum(relu_e * grad)
    loss = jnp.where(G > 0.0, loss, jnp.maximum(maxe, 0.0))
    out_ref[...] = jnp.full((1, 1, C), loss, dtype=jnp.float32)


@jax.jit
def kernel(logits, labels):
    lg = logits.reshape(B, R, C)
    lb = labels.reshape(B, R, C)
    out = pl.pallas_call(
        _loss_kernel,
        grid=(B,),
        in_specs=[
            pl.BlockSpec((1, R, C), lambda i: (i, 0, 0)),
            pl.BlockSpec((1, R, C), lambda i: (i, 0, 0)),
        ],
        out_specs=pl.BlockSpec((1, 1, C), lambda i: (i, 0, 0)),
        out_shape=jax.ShapeDtypeStruct((B, 1, C), jnp.float32),
    )(lg, lb)
    return jnp.mean(out[:, 0, 0])


# speedup vs baseline: 5.0675x; 1.4342x over previous
import functools

import jax
import jax.numpy as jnp
from jax.experimental import pallas as pl
from jax.experimental.pallas import tpu as pltpu

R = 2048
C = 128
N = R * C
LOGN = 18
B = 8


def _loss_kernel(logits_ref, labels_ref, out_ref):
    lg = logits_ref[...].reshape(R, C)
    lb = labels_ref[...].reshape(R, C)
    lbf = lb.astype(jnp.float32)
    signs = 2.0 * lbf - 1.0
    e = 1.0 - lg * signs

    maxe = jnp.max(e)

    bits = jax.lax.bitcast_convert_type(e, jnp.int32)
    ikey = bits ^ ((bits >> 31) & jnp.int32(0x7FFFFFFF))
    x = (ikey & jnp.int32(~1)) | lb

    idx = (jax.lax.broadcasted_iota(jnp.int32, (R, C), 1) * R
           + jax.lax.broadcasted_iota(jnp.int32, (R, C), 0))

    w = x ^ (-((idx >> 1) & 1))
    for kk in range(1, LOGN + 1):
        for j in range(kk - 1, -1, -1):
            s = 1 << j
            if s >= R:
                t = s // R
                liota = jax.lax.broadcasted_iota(jnp.int32, (1, C), 1)
                is_lower = (liota & t) == 0
                down = pltpu.roll(w, C - t, axis=1)
                hi = jnp.maximum(w, down)
                lo = jnp.minimum(w, down)
                w = jnp.where(is_lower, hi, pltpu.roll(lo, t, axis=1))
            elif s >= 8:
                O = R // (2 * s)
                y = w.reshape(O, 2, s, C)
                a = y[:, 0, :, :]
                b = y[:, 1, :, :]
                w = jnp.concatenate(
                    [jnp.maximum(a, b)[:, None], jnp.minimum(a, b)[:, None]],
                    axis=1).reshape(R, C)
            else:
                y = w.reshape(R // 8, 8, C)
                io8 = jax.lax.broadcasted_iota(jnp.int32, (1, 8, C), 1)
                is_lower = (io8 & s) == 0
                down = pltpu.roll(y, 8 - s, axis=1)
                hi = jnp.maximum(y, down)
                lo = jnp.minimum(y, down)
                w = jnp.where(is_lower, hi, pltpu.roll(lo, s, axis=1)).reshape(R, C)
        if kk < LOGN:
            w = w ^ (-(((idx >> kk) ^ (idx >> (kk + 1))) & 1))
    x = w

    gt = (x & 1).astype(jnp.float32)
    bdec = jnp.where(x >= 0, x, x ^ jnp.int32(0x7FFFFFFF))
    e_s = jax.lax.bitcast_convert_type(bdec, jnp.float32)
    relu_e = jnp.maximum(e_s, 0.0)

    col = gt
    riota = jax.lax.broadcasted_iota(jnp.int32, (R, C), 0)
    d = 1
    while d < R:
        shifted = pltpu.roll(col, d, axis=0)
        col = col + jnp.where(riota >= d, shifted, 0.0)
        d *= 2
    colsum = jax.lax.slice(col, (R - 1, 0), (R, C))
    liota = jax.lax.broadcasted_iota(jnp.int32, (1, C), 1)
    lane = colsum
    d = 1
    while d < C:
        shifted = pltpu.roll(lane, d, axis=1)
        lane = lane + jnp.where(liota >= d, shifted, 0.0)
        d *= 2
    G = jax.lax.slice(lane, (0, C - 1), (1, C))[0, 0]
    lane_ex = lane - colsum
    c = col + lane_ex

    kpos = idx.astype(jnp.float32)
    U = G + (kpos + 1.0) - c
    Um1 = jnp.maximum(U - 1.0 + gt, 1.0)
    grad = gt / U + (1.0 - gt) * (G - c) / (U * Um1)
    loss = jnp.sum(relu_e * grad)
    loss = jnp.where(G > 0.0, loss, jnp.maximum(maxe, 0.0))
    out_ref[...] = jnp.full((1, 1, C), loss, dtype=jnp.float32)


@jax.jit
def kernel(logits, labels):
    lg = logits.reshape(B, R, C)
    lb = labels.reshape(B, R, C)
    out = pl.pallas_call(
        _loss_kernel,
        grid=(B,),
        in_specs=[
            pl.BlockSpec((1, R, C), lambda i: (i, 0, 0)),
            pl.BlockSpec((1, R, C), lambda i: (i, 0, 0)),
        ],
        out_specs=pl.BlockSpec((1, 1, C), lambda i: (i, 0, 0)),
        out_shape=jax.ShapeDtypeStruct((B, 1, C), jnp.float32),
    )(lg, lb)
    return jnp.mean(out[:, 0, 0])
